# baseline (speedup 1.0000x reference)
import functools

import jax
import jax.numpy as jnp
from jax import lax
from jax.experimental import pallas as pl
from jax.experimental.pallas import tpu as pltpu
from jax.experimental.pallas import tpu_sc as plsc

_NUM_CORES = 2
_NUM_SUBCORES = 16
_NW = _NUM_CORES * _NUM_SUBCORES


@functools.partial(jax.jit, static_argnames=("chunk",))
def _sc_gather(table, idx, chunk=32):
    V, D = table.shape
    (B,) = idx.shape
    assert B % (8 * _NW) == 0
    b_per_w = B // _NW
    assert b_per_w % (2 * chunk) == 0
    n_rounds = b_per_w // (2 * chunk)

    mesh = plsc.VectorSubcoreMesh(core_axis_name="c", subcore_axis_name="s")

    @functools.partial(
        pl.kernel,
        mesh=mesh,
        out_type=jax.ShapeDtypeStruct((B, D), jnp.float32),
        scratch_types=[
            pltpu.VMEM((b_per_w,), jnp.int32),
            pltpu.VMEM((chunk, D), jnp.float32),
            pltpu.VMEM((chunk, D), jnp.float32),
            pltpu.SemaphoreType.DMA,
            pltpu.SemaphoreType.DMA,
            pltpu.SemaphoreType.DMA,
            pltpu.SemaphoreType.DMA,
        ],
    )
    def k(table_hbm, idx_hbm, out_hbm, idx_v, rows0, rows1, g0, g1, w0, w1):
        wid = lax.axis_index("s") * _NUM_CORES + lax.axis_index("c")
        base = wid * b_per_w
        pltpu.sync_copy(idx_hbm.at[pl.ds(base, b_per_w)], idx_v)

        rows = (rows0, rows1)
        gsem = (g0, g1)
        wsem = (w0, w1)

        def gather_desc(b, c):
            return pltpu.make_async_copy(
                table_hbm.at[idx_v.at[pl.ds(c * chunk, chunk)]], rows[b], gsem[b]
            )

        def write_desc(b, c):
            return pltpu.make_async_copy(
                rows[b], out_hbm.at[pl.ds(base + c * chunk, chunk)], wsem[b]
            )

        @pl.loop(0, n_rounds)
        def _(r):
            c0 = 2 * r
            for b in range(2):
                gather_desc(b, c0 + b).start()
            for b in range(2):
                gather_desc(b, c0 + b).wait()
        for b in range(2):
            write_desc(b, 2 * n_rounds - 2 + b).start()
            write_desc(b, 2 * n_rounds - 2 + b).wait()

    return k(table, idx)


def kernel(indices, embedding_weight):
    b, t = indices.shape
    _, d = embedding_weight.shape
    flat_idx = indices.reshape(-1).astype(jnp.int32)
    out = _sc_gather(embedding_weight, flat_idx)
    return out.reshape(b, t, d)


# speedup vs baseline: 1.2310x; 1.2310x over previous
import functools

import jax
import jax.numpy as jnp
from jax import lax
from jax.experimental import pallas as pl
from jax.experimental.pallas import tpu as pltpu
from jax.experimental.pallas import tpu_sc as plsc

_NUM_CORES = 2
_NUM_SUBCORES = 16
_NW = _NUM_CORES * _NUM_SUBCORES
_LAG = 8


@jax.jit
def _sc_lookup(table, idx):
    V, D = table.shape
    (B,) = idx.shape
    assert B % (8 * _NW) == 0
    b_per_w = B // _NW

    mesh = plsc.VectorSubcoreMesh(core_axis_name="c", subcore_axis_name="s")

    @functools.partial(
        pl.kernel,
        mesh=mesh,
        out_type=jax.ShapeDtypeStruct((B, D), jnp.float32),
        scratch_types=[
            pltpu.VMEM((V, D), jnp.float32),
            pltpu.VMEM((b_per_w,), jnp.int32),
            pltpu.SemaphoreType.DMA,
        ],
    )
    def k(table_hbm, idx_hbm, out_hbm, table_v, idx_v, wsem):
        wid = lax.axis_index("s") * _NUM_CORES + lax.axis_index("c")
        base = wid * b_per_w
        pltpu.sync_copy(table_hbm, table_v)
        pltpu.sync_copy(idx_hbm.at[pl.ds(base, b_per_w)], idx_v)

        def wait_row():
            pltpu.make_async_copy(table_v.at[0], out_hbm.at[base], wsem).wait()

        n_groups = b_per_w // 16

        @pl.loop(0, n_groups)
        def _(g):
            vec = idx_v[pl.ds(g * 16, 16)]
            j0 = base + g * 16
            for l in range(16):
                pltpu.async_copy(table_v.at[vec[l]], out_hbm.at[j0 + l], wsem)
            for _ in range(16):
                wait_row()

    return k(table, idx)


def kernel(indices, embedding_weight):
    b, t = indices.shape
    _, d = embedding_weight.shape
    flat_idx = indices.reshape(-1).astype(jnp.int32)
    out = _sc_lookup(embedding_weight, flat_idx)
    return out.reshape(b, t, d)
